# baseline (speedup 1.0000x reference)
import jax
import jax.numpy as jnp
from jax import lax
from jax.experimental import pallas as pl
from jax.experimental.pallas import tpu as pltpu

BN_EPS = 1e-5
COUT = 32
P_IN = 8
PC = 1
NB = 2
CHUNK = 512


def _rnd8(v):
    return ((v + 7) // 8) * 8


def _rnd16(v):
    return ((v + 15) // 16) * 16


def _cfg(H, W, cin):
    Ho, Wo = H // 2 + 1, W // 2 + 1
    Wp = _rnd16(max(W + P_IN + 1, 2 * Wo - 1 + PC))
    S = (H + 2) * Wp
    chunks = -(-(NB * S) // CHUNK)
    conv_rows = _rnd8(chunks * CHUNK + Wp + 8)
    in_rows = _rnd8(chunks * CHUNK + 2 * Wp + 16)
    return dict(H=H, W=W, cin=cin, Ho=Ho, Wo=Wo, Wp=Wp, S=S,
                chunks=chunks, conv_rows=conv_rows, in_rows=in_rows)


_LAYERS = (_cfg(84, 84, 4), _cfg(43, 43, 32), _cfg(22, 22, 32), _cfg(12, 12, 32))
_FEAT_STRIDE = 8 * _LAYERS[-1]['Ho']


def _zero(ref):
    rows, lanes = ref.shape
    z = jnp.zeros((CHUNK, lanes), ref.dtype)
    nfull = rows // CHUNK

    def body(i, c):
        ref[pl.ds(pl.multiple_of(i * CHUNK, 8), CHUNK), :] = z
        return c

    if nfull:
        lax.fori_loop(0, nfull, body, 0)
    rem = rows - nfull * CHUNK
    if rem:
        ref[pl.ds(nfull * CHUNK, rem), :] = jnp.zeros((rem, lanes), ref.dtype)


def _conv(load, w_ref, b_ref, conv_ref, cfg):
    Wp, chunks = cfg['Wp'], cfg['chunks']
    bias = b_ref[...]

    cin = w_ref.shape[1]
    slab_rows = CHUNK + 2 * Wp + 8

    def body(ci, c):
        base = pl.multiple_of(ci * CHUNK, 16)
        sb = load(base, slab_rows).astype(jnp.bfloat16)
        acc = jnp.zeros((CHUNK, COUT), jnp.float32)
        for dy in range(3):
            sd = lax.slice(sb, (dy * Wp, 0), (dy * Wp + CHUNK + 8, cin))
            for dx in range(3):
                o = dx + (P_IN - PC - 1)
                lhs = lax.slice(sd, (o, 0), (o + CHUNK, cin))
                acc = acc + jnp.dot(lhs, w_ref[dy * 3 + dx],
                                    preferred_element_type=jnp.float32)
        conv_ref[pl.ds(pl.multiple_of(base + Wp, 16), CHUNK), :] = \
            (acc + bias).astype(jnp.bfloat16)
        return c

    lax.fori_loop(0, chunks, body, 0)

    zrow = jnp.zeros((Wp, COUT), jnp.bfloat16)
    for b in range(NB):
        conv_ref[pl.ds(b * cfg['S'], Wp), :] = zrow
        if cfg['H'] % 2 == 0:
            conv_ref[pl.ds(b * cfg['S'] + (cfg['H'] + 1) * Wp, Wp), :] = zrow


def _pool(conv_ref, cfg, store):
    Ho, Wo, Wp, S, W = cfg['Ho'], cfg['Wo'], cfg['Wp'], cfg['S'], cfg['W']
    w2 = 2 * Wo
    j = lax.broadcasted_iota(jnp.int32, (Wo, w2), 0)
    k = lax.broadcasted_iota(jnp.int32, (Wo, w2), 1)
    sel_l = jnp.logical_and(k == 2 * j, k > 0).astype(jnp.bfloat16)
    rok = k == 2 * j + 1
    if W % 2 == 0:
        rok = jnp.logical_and(rok, k < w2 - 1)
    sel_r = rok.astype(jnp.bfloat16)

    def body(i, c):
        b = i // Ho
        r = i - b * Ho
        base = pl.multiple_of(b * S + 2 * r * Wp, 16)
        slab = conv_ref[pl.ds(base, Wp + w2), :]
        m = jnp.maximum(lax.slice(slab, (0, 0), (w2, COUT)),
                        lax.slice(slab, (Wp, 0), (Wp + w2, COUT)))
        left = jnp.dot(sel_l, m, preferred_element_type=jnp.float32)
        right = jnp.dot(sel_r, m, preferred_element_type=jnp.float32)
        store(b, r, jnp.maximum(jnp.maximum(left, right), 0.0).astype(jnp.bfloat16))
        return c

    lax.fori_loop(0, NB * Ho, body, 0)


def _fwd(x_ref, w1, b1, w2, b2, w3, b3, w4, b4, wfc, fcb, o_ref,
         cv1, in2, cv2, in3, cv3, in4, cv4, feat):
    cfg1, cfg2, cfg3, cfg4 = _LAYERS
    for ref in (in2, in3, in4, feat):
        _zero(ref)

    ins = (None, in2, in3, in4)
    cvs = (cv1, cv2, cv3, cv4)
    ws = (w1, w2, w3, w4)
    bs = (b1, b2, b3, b4)

    def mk_store(in_ref, cfg_next):
        Wpn, Sn = cfg_next['Wp'], cfg_next['S']

        def st(b, r, v):
            dst = pl.multiple_of(b * Sn + (r + 1) * Wpn + P_IN, 8)
            in_ref[pl.ds(dst, v.shape[0]), :] = v.astype(jnp.float32)
        return st

    def feat_store(b, r, v):
        feat[pl.ds(pl.multiple_of(b * _FEAT_STRIDE + 8 * r, 8), v.shape[0]), :] = \
            v.astype(jnp.float32)

    for li, cfg in enumerate(_LAYERS):
        if li == 0:
            load = lambda o, n: x_ref[0, pl.ds(o, n), :]
        else:
            in_ref = ins[li]
            load = lambda o, n, _r=in_ref: _r[pl.ds(o, n), :]
        _conv(load, ws[li], bs[li], cvs[li], cfg)
        if li + 1 < len(_LAYERS):
            store = mk_store(ins[li + 1], _LAYERS[li + 1])
        else:
            store = feat_store
        _pool(cvs[li], cfg, store)

    nc = wfc.shape[0]
    lane = lax.broadcasted_iota(jnp.int32, (1, nc), 1)
    bvec = fcb[...]
    wfs = [wfc[n].astype(jnp.float32) for n in range(nc)]
    for b in range(NB):
        f = feat[pl.ds(b * _FEAT_STRIDE, _FEAT_STRIDE), :]
        logits = bvec
        for n in range(nc):
            logits = logits + jnp.where(lane == n, jnp.sum(f * wfs[n]), 0.0)
        o_ref[0, pl.ds(b, 1), :] = logits


def _fold_bn(w, b, gamma, beta, mean, var):
    s = gamma / jnp.sqrt(var + BN_EPS)
    return w * s[:, None, None, None], (b - mean) * s + beta


def kernel(x, w1, b1, gamma1, beta1, mean1, var1,
           w2, b2, gamma2, beta2, mean2, var2,
           w3, b3, gamma3, beta3, mean3, var3,
           w4, b4, gamma4, beta4, mean4, var4,
           fc_w, fc_b):
    cfg1 = _LAYERS[0]
    B = x.shape[0]
    G = B // NB

    xh = jnp.transpose(x, (0, 2, 3, 1))
    xh = jnp.pad(xh, ((0, 0), (1, 1), (P_IN, cfg1['Wp'] - P_IN - cfg1['W']),
                      (0, cfg1['cin'] - xh.shape[-1])))
    xh = xh.reshape(B, cfg1['S'], cfg1['cin'])
    xh = xh.reshape(G, NB * cfg1['S'], cfg1['cin'])
    xh = jnp.pad(xh, ((0, 0), (0, cfg1['in_rows'] - NB * cfg1['S']), (0, 0)))
    xh = xh.astype(jnp.bfloat16)

    inputs = [xh]
    in_specs = [pl.BlockSpec((1, cfg1['in_rows'], cfg1['cin']),
                             lambda g: (g, 0, 0))]
    assert cfg1['in_rows'] % 16 == 0
    raw = [(w1, b1, gamma1, beta1, mean1, var1),
           (w2, b2, gamma2, beta2, mean2, var2),
           (w3, b3, gamma3, beta3, mean3, var3),
           (w4, b4, gamma4, beta4, mean4, var4)]
    for cfg, prm in zip(_LAYERS, raw):
        w, bias = _fold_bn(*prm)
        cin = w.shape[1]
        if cin < cfg['cin']:
            w = jnp.pad(w, ((0, 0), (0, cfg['cin'] - cin), (0, 0), (0, 0)))
        w9 = jnp.transpose(w, (2, 3, 1, 0)).reshape(9, cfg['cin'], COUT)
        inputs += [w9.astype(jnp.bfloat16), bias.reshape(1, COUT)]
        in_specs += [pl.BlockSpec((9, cfg['cin'], COUT), lambda g: (0, 0, 0)),
                     pl.BlockSpec((1, COUT), lambda g: (0, 0))]

    nc = fc_w.shape[0]
    ho, wo = _LAYERS[-1]['Ho'], _LAYERS[-1]['Wo']
    wfc = fc_w.reshape(nc, COUT, ho, wo)
    wfc = jnp.transpose(wfc, (0, 2, 3, 1))
    wfc = jnp.pad(wfc, ((0, 0), (0, 0), (0, 8 - wo), (0, 0)))
    wfc = wfc.reshape(nc, _FEAT_STRIDE, COUT).astype(jnp.bfloat16)
    inputs += [wfc, fc_b.reshape(1, nc)]
    in_specs += [pl.BlockSpec((nc, _FEAT_STRIDE, COUT), lambda g: (0, 0, 0)),
                 pl.BlockSpec((1, nc), lambda g: (0, 0))]

    scratch = [pltpu.VMEM((_LAYERS[0]['conv_rows'], COUT), jnp.bfloat16)]
    for cfg in _LAYERS[1:]:
        scratch.append(pltpu.VMEM((cfg['in_rows'], cfg['cin']), jnp.float32))
        scratch.append(pltpu.VMEM((cfg['conv_rows'], COUT), jnp.bfloat16))
    scratch.append(pltpu.VMEM((NB * _FEAT_STRIDE, COUT), jnp.float32))

    out = pl.pallas_call(
        _fwd,
        grid=(G,),
        in_specs=in_specs,
        out_specs=pl.BlockSpec((1, NB, nc), lambda g: (g, 0, 0)),
        out_shape=jax.ShapeDtypeStruct((G, NB, nc), jnp.float32),
        scratch_shapes=scratch,
        compiler_params=pltpu.CompilerParams(
            dimension_semantics=("parallel",)),
    )(*inputs)
    return out.reshape(B, nc)


# speedup vs baseline: 1.3298x; 1.3298x over previous
import jax
import jax.numpy as jnp
from jax import lax
from jax.experimental import pallas as pl
from jax.experimental.pallas import tpu as pltpu

BN_EPS = 1e-5
COUT = 32
P_IN = 8
PC = 1
NB = 2
CHUNK = 512


def _rnd8(v):
    return ((v + 7) // 8) * 8


def _rnd16(v):
    return ((v + 15) // 16) * 16


def _cfg(H, W, cin):
    Ho, Wo = H // 2 + 1, W // 2 + 1
    Wp = _rnd16(max(W + P_IN + 1, 2 * Wo - 1 + PC))
    S = (H + 2) * Wp
    chunks = -(-(NB * S) // CHUNK)
    conv_rows = _rnd8(chunks * CHUNK + Wp + 8)
    in_rows = _rnd8(chunks * CHUNK + 2 * Wp + 16)
    return dict(H=H, W=W, cin=cin, Ho=Ho, Wo=Wo, Wp=Wp, S=S,
                chunks=chunks, conv_rows=conv_rows, in_rows=in_rows)


_LAYERS = (_cfg(84, 84, 4), _cfg(43, 43, 32), _cfg(22, 22, 32), _cfg(12, 12, 32))
_FEAT_STRIDE = 8 * _LAYERS[-1]['Ho']


def _zero(ref):
    rows, lanes = ref.shape
    z = jnp.zeros((CHUNK, lanes), ref.dtype)
    nfull = rows // CHUNK

    def body(i, c):
        ref[pl.ds(pl.multiple_of(i * CHUNK, 8), CHUNK), :] = z
        return c

    if nfull:
        lax.fori_loop(0, nfull, body, 0)
    rem = rows - nfull * CHUNK
    if rem:
        ref[pl.ds(nfull * CHUNK, rem), :] = jnp.zeros((rem, lanes), ref.dtype)


def _conv(load, w_ref, b_ref, conv_ref, cfg):
    Wp, chunks = cfg['Wp'], cfg['chunks']
    bias = b_ref[...]

    cin = w_ref.shape[1]
    slab_rows = CHUNK + 2 * Wp + 8

    def body(ci, c):
        base = pl.multiple_of(ci * CHUNK, 16)
        sb = load(base, slab_rows).astype(jnp.bfloat16)
        acc = jnp.zeros((CHUNK, COUT), jnp.float32)
        for dy in range(3):
            sd = lax.slice(sb, (dy * Wp, 0), (dy * Wp + CHUNK + 8, cin))
            for dx in range(3):
                o = dx + (P_IN - PC - 1)
                lhs = lax.slice(sd, (o, 0), (o + CHUNK, cin))
                acc = acc + jnp.dot(lhs, w_ref[dy * 3 + dx],
                                    preferred_element_type=jnp.float32)
        conv_ref[pl.ds(pl.multiple_of(base + Wp, 16), CHUNK), :] = \
            (acc + bias).astype(jnp.bfloat16)
        return c

    lax.fori_loop(0, chunks, body, 0)

    zrow = jnp.zeros((Wp, COUT), jnp.bfloat16)
    for b in range(NB):
        conv_ref[pl.ds(b * cfg['S'], Wp), :] = zrow
        if cfg['H'] % 2 == 0:
            conv_ref[pl.ds(b * cfg['S'] + (cfg['H'] + 1) * Wp, Wp), :] = zrow


def _pool(conv_ref, cfg, store):
    Ho, Wo, Wp, S, W = cfg['Ho'], cfg['Wo'], cfg['Wp'], cfg['S'], cfg['W']
    w2 = 2 * Wo
    j = lax.broadcasted_iota(jnp.int32, (Wo, w2), 0)
    k = lax.broadcasted_iota(jnp.int32, (Wo, w2), 1)
    sel_l = jnp.logical_and(k == 2 * j, k > 0).astype(jnp.bfloat16)
    rok = k == 2 * j + 1
    if W % 2 == 0:
        rok = jnp.logical_and(rok, k < w2 - 1)
    sel_r = rok.astype(jnp.bfloat16)

    def body(i, c):
        b = i // Ho
        r = i - b * Ho
        base = pl.multiple_of(b * S + 2 * r * Wp, 16)
        slab = conv_ref[pl.ds(base, Wp + w2), :]
        m = jnp.maximum(lax.slice(slab, (0, 0), (w2, COUT)),
                        lax.slice(slab, (Wp, 0), (Wp + w2, COUT)))
        left = jnp.dot(sel_l, m, preferred_element_type=jnp.float32)
        right = jnp.dot(sel_r, m, preferred_element_type=jnp.float32)
        store(b, r, jnp.maximum(jnp.maximum(left, right), 0.0).astype(jnp.bfloat16))
        return c

    lax.fori_loop(0, NB * Ho, body, 0)


def _fwd(x_ref, w1, b1, w2, b2, w3, b3, w4, b4, wfc, fcb, o_ref,
         in1, cv1, in2, cv2, in3, cv3, in4, cv4, feat):
    cfg1, cfg2, cfg3, cfg4 = _LAYERS
    for ref in (in1, in2, in3, in4, feat):
        _zero(ref)

    W1, Wp1, S1 = cfg1['W'], cfg1['Wp'], cfg1['S']
    HW = cfg1['H'] * W1

    def copy_body(i, c):
        b = i // cfg1['H']
        r = i - b * cfg1['H']
        dst = pl.multiple_of(b * S1 + (r + 1) * Wp1 + P_IN, 8)
        in1[pl.ds(dst, W1), :] = x_ref[0, pl.ds(b * HW + r * W1, W1), :]
        return c

    lax.fori_loop(0, NB * cfg1['H'], copy_body, 0)

    ins = (in1, in2, in3, in4)
    cvs = (cv1, cv2, cv3, cv4)
    ws = (w1, w2, w3, w4)
    bs = (b1, b2, b3, b4)

    def mk_store(in_ref, cfg_next):
        Wpn, Sn = cfg_next['Wp'], cfg_next['S']

        def st(b, r, v):
            dst = pl.multiple_of(b * Sn + (r + 1) * Wpn + P_IN, 8)
            in_ref[pl.ds(dst, v.shape[0]), :] = v.astype(jnp.float32)
        return st

    def feat_store(b, r, v):
        feat[pl.ds(pl.multiple_of(b * _FEAT_STRIDE + 8 * r, 8), v.shape[0]), :] = \
            v.astype(jnp.float32)

    for li, cfg in enumerate(_LAYERS):
        in_ref = ins[li]
        load = lambda o, n, _r=in_ref: _r[pl.ds(o, n), :]
        _conv(load, ws[li], bs[li], cvs[li], cfg)
        if li + 1 < len(_LAYERS):
            store = mk_store(ins[li + 1], _LAYERS[li + 1])
        else:
            store = feat_store
        _pool(cvs[li], cfg, store)

    nc = wfc.shape[0]
    lane = lax.broadcasted_iota(jnp.int32, (1, nc), 1)
    bvec = fcb[...]
    wfs = [wfc[n].astype(jnp.float32) for n in range(nc)]
    for b in range(NB):
        f = feat[pl.ds(b * _FEAT_STRIDE, _FEAT_STRIDE), :]
        logits = bvec
        for n in range(nc):
            logits = logits + jnp.where(lane == n, jnp.sum(f * wfs[n]), 0.0)
        o_ref[0, pl.ds(b, 1), :] = logits


def _fold_bn(w, b, gamma, beta, mean, var):
    s = gamma / jnp.sqrt(var + BN_EPS)
    return w * s[:, None, None, None], (b - mean) * s + beta


def kernel(x, w1, b1, gamma1, beta1, mean1, var1,
           w2, b2, gamma2, beta2, mean2, var2,
           w3, b3, gamma3, beta3, mean3, var3,
           w4, b4, gamma4, beta4, mean4, var4,
           fc_w, fc_b):
    cfg1 = _LAYERS[0]
    B = x.shape[0]
    G = B // NB

    xh = jnp.transpose(x, (0, 2, 3, 1))
    xh = jnp.pad(xh, ((0, 0), (0, 0), (0, 0),
                      (0, cfg1['cin'] - xh.shape[-1])))
    xh = xh.reshape(G, NB * cfg1['H'] * cfg1['W'], cfg1['cin'])

    inputs = [xh]
    in_specs = [pl.BlockSpec((1, NB * cfg1['H'] * cfg1['W'], cfg1['cin']),
                             lambda g: (g, 0, 0))]
    raw = [(w1, b1, gamma1, beta1, mean1, var1),
           (w2, b2, gamma2, beta2, mean2, var2),
           (w3, b3, gamma3, beta3, mean3, var3),
           (w4, b4, gamma4, beta4, mean4, var4)]
    for cfg, prm in zip(_LAYERS, raw):
        w, bias = _fold_bn(*prm)
        cin = w.shape[1]
        if cin < cfg['cin']:
            w = jnp.pad(w, ((0, 0), (0, cfg['cin'] - cin), (0, 0), (0, 0)))
        w9 = jnp.transpose(w, (2, 3, 1, 0)).reshape(9, cfg['cin'], COUT)
        inputs += [w9.astype(jnp.bfloat16), bias.reshape(1, COUT)]
        in_specs += [pl.BlockSpec((9, cfg['cin'], COUT), lambda g: (0, 0, 0)),
                     pl.BlockSpec((1, COUT), lambda g: (0, 0))]

    nc = fc_w.shape[0]
    ho, wo = _LAYERS[-1]['Ho'], _LAYERS[-1]['Wo']
    wfc = fc_w.reshape(nc, COUT, ho, wo)
    wfc = jnp.transpose(wfc, (0, 2, 3, 1))
    wfc = jnp.pad(wfc, ((0, 0), (0, 0), (0, 8 - wo), (0, 0)))
    wfc = wfc.reshape(nc, _FEAT_STRIDE, COUT).astype(jnp.bfloat16)
    inputs += [wfc, fc_b.reshape(1, nc)]
    in_specs += [pl.BlockSpec((nc, _FEAT_STRIDE, COUT), lambda g: (0, 0, 0)),
                 pl.BlockSpec((1, nc), lambda g: (0, 0))]

    scratch = [pltpu.VMEM((_LAYERS[0]['in_rows'], _LAYERS[0]['cin']), jnp.float32),
               pltpu.VMEM((_LAYERS[0]['conv_rows'], COUT), jnp.bfloat16)]
    for cfg in _LAYERS[1:]:
        scratch.append(pltpu.VMEM((cfg['in_rows'], cfg['cin']), jnp.float32))
        scratch.append(pltpu.VMEM((cfg['conv_rows'], COUT), jnp.bfloat16))
    scratch.append(pltpu.VMEM((NB * _FEAT_STRIDE, COUT), jnp.float32))

    out = pl.pallas_call(
        _fwd,
        grid=(G,),
        in_specs=in_specs,
        out_specs=pl.BlockSpec((1, NB, nc), lambda g: (g, 0, 0)),
        out_shape=jax.ShapeDtypeStruct((G, NB, nc), jnp.float32),
        scratch_shapes=scratch,
        compiler_params=pltpu.CompilerParams(
            dimension_semantics=("parallel",)),
    )(*inputs)
    return out.reshape(B, nc)
